# baseline (speedup 1.0000x reference)
import functools

import jax
import jax.numpy as jnp
from jax import lax
from jax.experimental import pallas as pl
from jax.experimental.pallas import tpu as pltpu
from jax.experimental.pallas import tpu_sc as plsc

_N, _D, _E = 50000, 256, 100000
_NC, _NS = 2, 16
_NW = _NC * _NS
_B = 40
_NCHT = _E // _B
_T = 80
_CHB = 4 * _B
_CROWS = 80
_NCOPY = _N // _CROWS
_VT = 20

_mesh = plsc.VectorSubcoreMesh(core_axis_name="c", subcore_axis_name="s")


@functools.partial(
    pl.kernel,
    out_type=jax.ShapeDtypeStruct((_E, _D), jnp.float32),
    mesh=_mesh,
    scratch_types=[
        [pltpu.VMEM((_CHB,), jnp.int32) for _ in range(4)],
        [pltpu.VMEM((_CHB, _D // 2), jnp.float32) for _ in range(4)],
        [pltpu.VMEM((_B, _D), jnp.float32) for _ in range(2)],
        [pltpu.SemaphoreType.DMA for _ in range(4)],
        [pltpu.SemaphoreType.DMA for _ in range(4)],
        [pltpu.SemaphoreType.DMA for _ in range(2)],
    ],
    compiler_params=pltpu.CompilerParams(use_tc_tiling_on_sc=False),
)
def _graph_pool(x_hbm, idx_hbm, out_hbm, ib, gb, rb, isem, gsem, ssem):
    w = lax.axis_index("s") * _NC + lax.axis_index("c")

    def valid(t):
        return w + t * _NW < _NCHT

    def idx_copy(t, k):
        return pltpu.make_async_copy(idx_hbm.at[w + t * _NW], ib[k], isem[k])

    def gather_copy(k):
        return pltpu.make_async_copy(x_hbm.at[ib[k]], gb[k], gsem[k])

    def scatter_copy(t, k2):
        base = (w + t * _NW) * _B
        return pltpu.make_async_copy(rb[k2], out_hbm.at[pl.ds(base, _B)],
                                     ssem[k2])

    def issue_idx(t, k):
        @pl.when(valid(t))
        def _():
            idx_copy(t, k).start()

    def wait_idx(t, k):
        @pl.when(valid(t))
        def _():
            idx_copy(t, k).wait()

    def issue_gather(t, k):
        @pl.when(valid(t))
        def _():
            gather_copy(k).start()

    def wait_gather(t, k):
        @pl.when(valid(t))
        def _():
            gather_copy(k).wait()

    def issue_scatter(t, k2):
        @pl.when(valid(t))
        def _():
            scatter_copy(t, k2).start()

    def wait_scatter(t, k2):
        @pl.when((t >= 0) & valid(t))
        def _():
            scatter_copy(t, k2).wait()

    def compute(t, k, k2):
        @pl.when(valid(t))
        def _():
            src, dst = gb[k], rb[k2]

            def row_body(j, rc):
                for q in range(_D // 32):
                    v0 = src[4 * j, pl.ds(q * 16, 16)]
                    v1 = src[4 * j + 2, pl.ds(q * 16, 16)]
                    dst[j, pl.ds(q * 16, 16)] = (v0 + v1) * 0.5
                for q in range(_D // 32):
                    v0 = src[4 * j + 1, pl.ds(q * 16, 16)]
                    v1 = src[4 * j + 3, pl.ds(q * 16, 16)]
                    dst[j, pl.ds(_D // 2 + q * 16, 16)] = (v0 + v1) * 0.5
                return rc

            lax.fori_loop(0, _B, row_body, 0, unroll=False)

    for t in range(3):
        issue_idx(t, t)
    for t in range(3):
        wait_idx(t, t)
        issue_gather(t, t)
    issue_idx(3, 3)

    def step(u, carry):
        for k in range(4):
            t = 4 * u + k
            k3 = (k + 3) % 4
            k2 = k % 2
            wait_gather(t, k)
            issue_idx(t + 4, k)
            wait_idx(t + 3, k3)
            issue_gather(t + 3, k3)
            wait_scatter(t - 2, k2)
            compute(t, k, k2)
            issue_scatter(t, k2)
        return carry

    lax.fori_loop(0, _T // 4, step, 0, unroll=False)

    wait_scatter(_T - 2, 0)
    wait_scatter(_T - 1, 1)


def kernel(inputs, pool_idx):
    idx2 = pool_idx.reshape(_E, 2).astype(jnp.int32) * 2
    idx4 = (idx2[:, :, None] + jnp.arange(2, dtype=jnp.int32)).reshape(
        _NCHT, _CHB)
    mid = _graph_pool(inputs.reshape(2 * _N, _D // 2), idx4)
    return jnp.concatenate([inputs, mid], axis=0)


# speedup vs baseline: 1.4480x; 1.4480x over previous
import functools

import jax
import jax.numpy as jnp
from jax import lax
from jax.experimental import pallas as pl
from jax.experimental.pallas import tpu as pltpu
from jax.experimental.pallas import tpu_sc as plsc

_N, _D, _E = 50000, 256, 100000
_NC, _NS = 2, 16
_NW = _NC * _NS
_B = 40
_NCHT = _E // _B
_T = 80
_CHB = 2 * _B
_CROWS = 80
_NCOPY = _N // _CROWS
_VT = 20

_mesh = plsc.VectorSubcoreMesh(core_axis_name="c", subcore_axis_name="s")


@functools.partial(
    pl.kernel,
    out_type=jax.ShapeDtypeStruct((_E, _D), jnp.float32),
    mesh=_mesh,
    scratch_types=[
        [pltpu.VMEM((_CHB,), jnp.int32) for _ in range(4)],
        [pltpu.VMEM((_CHB, _D), jnp.float32) for _ in range(4)],
        [pltpu.VMEM((_B, _D), jnp.float32) for _ in range(2)],
        [pltpu.SemaphoreType.DMA for _ in range(4)],
        [pltpu.SemaphoreType.DMA for _ in range(4)],
        [pltpu.SemaphoreType.DMA for _ in range(2)],
    ],
    compiler_params=pltpu.CompilerParams(use_tc_tiling_on_sc=False),
)
def _graph_pool(x_hbm, idx_hbm, out_hbm, ib, gb, rb, isem, gsem, ssem):
    w = lax.axis_index("s") * _NC + lax.axis_index("c")

    def valid(t):
        return w + t * _NW < _NCHT

    def idx_copy(t, k):
        return pltpu.make_async_copy(idx_hbm.at[w + t * _NW], ib[k], isem[k])

    def gather_copy(k):
        return pltpu.make_async_copy(x_hbm.at[ib[k]], gb[k], gsem[k])

    def scatter_copy(t, k2):
        base = (w + t * _NW) * _B
        return pltpu.make_async_copy(rb[k2], out_hbm.at[pl.ds(base, _B)],
                                     ssem[k2])

    def issue_idx(t, k):
        @pl.when(valid(t))
        def _():
            idx_copy(t, k).start()

    def wait_idx(t, k):
        @pl.when(valid(t))
        def _():
            idx_copy(t, k).wait()

    def issue_gather(t, k):
        @pl.when(valid(t))
        def _():
            gather_copy(k).start()

    def wait_gather(t, k):
        @pl.when(valid(t))
        def _():
            gather_copy(k).wait()

    def issue_scatter(t, k2):
        @pl.when(valid(t))
        def _():
            scatter_copy(t, k2).start()

    def wait_scatter(t, k2):
        @pl.when((t >= 0) & valid(t))
        def _():
            scatter_copy(t, k2).wait()

    def compute(t, k, k2):
        @pl.when(valid(t))
        def _():
            src, dst = gb[k], rb[k2]

            def row_body(j, rc):
                for q in range(_D // 16):
                    v0 = src[2 * j, pl.ds(q * 16, 16)]
                    v1 = src[2 * j + 1, pl.ds(q * 16, 16)]
                    dst[j, pl.ds(q * 16, 16)] = (v0 + v1) * 0.5
                return rc

            lax.fori_loop(0, _B, row_body, 0, unroll=False)

    for t in range(3):
        issue_idx(t, t)
    for t in range(3):
        wait_idx(t, t)
        issue_gather(t, t)
    issue_idx(3, 3)

    def step(u, carry):
        for k in range(4):
            t = 4 * u + k
            k3 = (k + 3) % 4
            k2 = k % 2
            wait_gather(t, k)
            issue_idx(t + 4, k)
            wait_idx(t + 3, k3)
            issue_gather(t + 3, k3)
            wait_scatter(t - 2, k2)
            compute(t, k, k2)
            issue_scatter(t, k2)
        return carry

    lax.fori_loop(0, _T // 4, step, 0, unroll=False)

    wait_scatter(_T - 2, 0)
    wait_scatter(_T - 1, 1)


def kernel(inputs, pool_idx):
    idx = pool_idx.reshape(_NCHT, _CHB).astype(jnp.int32)
    mid = _graph_pool(inputs, idx)
    return jnp.concatenate([inputs, mid], axis=0)


# speedup vs baseline: 1.6742x; 1.1562x over previous
import functools

import jax
import jax.numpy as jnp
from jax import lax
from jax.experimental import pallas as pl
from jax.experimental.pallas import tpu as pltpu
from jax.experimental.pallas import tpu_sc as plsc

_N, _D, _E = 50000, 256, 100000
_NC, _NS = 2, 16
_NW = _NC * _NS
_B = 40
_NCHT = _E // _B
_T = 80
_CHB = 2 * _B
_NBLK = _B * _D // 1024
_MIDI = _E * _D // 1024

_mesh = plsc.VectorSubcoreMesh(core_axis_name="c", subcore_axis_name="s")


@functools.partial(
    pl.kernel,
    out_type=jax.ShapeDtypeStruct((_MIDI, 8, 128), jnp.float32),
    mesh=_mesh,
    scratch_types=[
        [pltpu.VMEM((_CHB,), jnp.int32) for _ in range(4)],
        [pltpu.VMEM((_CHB, _D), jnp.float32) for _ in range(4)],
        [pltpu.VMEM((_NBLK, 8, 128), jnp.float32) for _ in range(2)],
        [pltpu.SemaphoreType.DMA for _ in range(4)],
        [pltpu.SemaphoreType.DMA for _ in range(4)],
        [pltpu.SemaphoreType.DMA for _ in range(2)],
    ],
    compiler_params=pltpu.CompilerParams(use_tc_tiling_on_sc=False),
)
def _sc_edges(x_hbm, idx_hbm, mid_hbm, ib, gb, rb, isem, gsem, ssem):
    w = lax.axis_index("s") * _NC + lax.axis_index("c")

    def valid(t):
        return w + t * _NW < _NCHT

    def idx_copy(t, k):
        return pltpu.make_async_copy(idx_hbm.at[w + t * _NW], ib[k], isem[k])

    def gather_copy(k):
        return pltpu.make_async_copy(x_hbm.at[ib[k]], gb[k], gsem[k])

    def scatter_copy(t, k2):
        base = (w + t * _NW) * _NBLK
        return pltpu.make_async_copy(rb[k2], mid_hbm.at[pl.ds(base, _NBLK)],
                                     ssem[k2])

    def issue_idx(t, k):
        @pl.when(valid(t))
        def _():
            idx_copy(t, k).start()

    def wait_idx(t, k):
        @pl.when(valid(t))
        def _():
            idx_copy(t, k).wait()

    def issue_gather(t, k):
        @pl.when(valid(t))
        def _():
            gather_copy(k).start()

    def wait_gather(t, k):
        @pl.when(valid(t))
        def _():
            gather_copy(k).wait()

    def issue_scatter(t, k2):
        @pl.when(valid(t))
        def _():
            scatter_copy(t, k2).start()

    def wait_scatter(t, k2):
        @pl.when((t >= 0) & valid(t))
        def _():
            scatter_copy(t, k2).wait()

    def compute(t, k, k2):
        @pl.when(valid(t))
        def _():
            src, dst = gb[k], rb[k2]

            def row_body(j, rc):
                jhi = j // 8
                jlo = j % 8
                for q in range(_D // 16):
                    v0 = src[2 * j, pl.ds(q * 16, 16)]
                    v1 = src[2 * j + 1, pl.ds(q * 16, 16)]
                    dst[2 * jhi + q // 8, jlo, pl.ds((q % 8) * 16, 16)] = (
                        (v0 + v1) * 0.5)
                return rc

            lax.fori_loop(0, _B, row_body, 0, unroll=False)

    for t in range(3):
        issue_idx(t, t)
    for t in range(3):
        wait_idx(t, t)
        issue_gather(t, t)
    issue_idx(3, 3)

    def step(u, carry):
        for k in range(4):
            t = 4 * u + k
            k3 = (k + 3) % 4
            k2 = k % 2
            wait_gather(t, k)
            issue_idx(t + 4, k)
            wait_idx(t + 3, k3)
            issue_gather(t + 3, k3)
            wait_scatter(t - 2, k2)
            compute(t, k, k2)
            issue_scatter(t, k2)
        return carry

    lax.fori_loop(0, _T // 4, step, 0, unroll=False)

    wait_scatter(_T - 2, 0)
    wait_scatter(_T - 1, 1)


_TOPR = 1000


def _tc_top_body(x_ref, o_ref):
    o_ref[...] = x_ref[...]


_tc_top = pl.pallas_call(
    _tc_top_body,
    grid=(_N // _TOPR,),
    in_specs=[pl.BlockSpec((_TOPR, _D), lambda i: (i, 0))],
    out_specs=pl.BlockSpec((_TOPR, _D), lambda i: (i, 0)),
    out_shape=jax.ShapeDtypeStruct((_N + _E, _D), jnp.float32),
)

_MBI = 250


def _tc_mid_body(o_in_ref, m_ref, o_ref):
    del o_in_ref
    m = m_ref[...]
    m = m.reshape(_MBI // 2, 2, 8, 128)
    m = jnp.swapaxes(m, 1, 2)
    o_ref[...] = m.reshape(_MBI * 4, _D)


_tc_mid = pl.pallas_call(
    _tc_mid_body,
    grid=(_MIDI // _MBI,),
    in_specs=[
        pl.BlockSpec((8, 128), lambda i: (0, 0)),
        pl.BlockSpec((_MBI, 8, 128), lambda i: (i, 0, 0)),
    ],
    out_specs=pl.BlockSpec((_MBI * 4, _D), lambda i: (_N // (_MBI * 4) + i, 0)),
    out_shape=jax.ShapeDtypeStruct((_N + _E, _D), jnp.float32),
    input_output_aliases={0: 0},
)


def kernel(inputs, pool_idx):
    idx = pool_idx.reshape(_NCHT, _CHB).astype(jnp.int32)
    mid3 = _sc_edges(inputs, idx)
    out0 = _tc_top(inputs)
    return _tc_mid(out0, mid3)


# speedup vs baseline: 1.7591x; 1.0507x over previous
import functools

import jax
import jax.numpy as jnp
from jax import lax
from jax.experimental import pallas as pl
from jax.experimental.pallas import tpu as pltpu
from jax.experimental.pallas import tpu_sc as plsc

_N, _D, _E = 50000, 256, 100000
_NC, _NS = 2, 16
_NW = _NC * _NS
_B = 40
_NCHT = _E // _B
_T = 80
_CHB = 2 * _B
_NBLK = _B * _D // 1024
_MIDI = _E * _D // 1024

_mesh = plsc.VectorSubcoreMesh(core_axis_name="c", subcore_axis_name="s")


@functools.partial(
    pl.kernel,
    out_type=jax.ShapeDtypeStruct((_MIDI, 8, 128), jnp.float32),
    mesh=_mesh,
    scratch_types=[
        [pltpu.VMEM((_CHB,), jnp.int32) for _ in range(4)],
        [pltpu.VMEM((_CHB, _D), jnp.float32) for _ in range(4)],
        [pltpu.VMEM((_NBLK, 8, 128), jnp.float32) for _ in range(2)],
        [pltpu.SemaphoreType.DMA for _ in range(4)],
        [pltpu.SemaphoreType.DMA for _ in range(4)],
        [pltpu.SemaphoreType.DMA for _ in range(2)],
    ],
    compiler_params=pltpu.CompilerParams(use_tc_tiling_on_sc=False),
)
def _sc_edges(x_hbm, idx_hbm, mid_hbm, ib, gb, rb, isem, gsem, ssem):
    w = lax.axis_index("s") * _NC + lax.axis_index("c")

    def valid(t):
        return w + t * _NW < _NCHT

    def idx_copy(t, k):
        return pltpu.make_async_copy(idx_hbm.at[w + t * _NW], ib[k], isem[k])

    def gather_copy(k):
        return pltpu.make_async_copy(x_hbm.at[ib[k]], gb[k], gsem[k])

    def scatter_copy(t, k2):
        base = (w + t * _NW) * _NBLK
        return pltpu.make_async_copy(rb[k2], mid_hbm.at[pl.ds(base, _NBLK)],
                                     ssem[k2])

    def issue_idx(t, k):
        @pl.when(valid(t))
        def _():
            idx_copy(t, k).start()

    def wait_idx(t, k):
        @pl.when(valid(t))
        def _():
            idx_copy(t, k).wait()

    def issue_gather(t, k):
        @pl.when(valid(t))
        def _():
            gather_copy(k).start()

    def wait_gather(t, k):
        @pl.when(valid(t))
        def _():
            gather_copy(k).wait()

    def issue_scatter(t, k2):
        @pl.when(valid(t))
        def _():
            scatter_copy(t, k2).start()

    def wait_scatter(t, k2):
        @pl.when((t >= 0) & valid(t))
        def _():
            scatter_copy(t, k2).wait()

    def compute(t, k, k2):
        @pl.when(valid(t))
        def _():
            src, dst = gb[k], rb[k2]

            def row_body(j, rc):
                jhi = j // 8
                jlo = j % 8
                for q in range(_D // 16):
                    v0 = src[2 * j, pl.ds(q * 16, 16)]
                    v1 = src[2 * j + 1, pl.ds(q * 16, 16)]
                    dst[2 * jhi + q // 8, jlo, pl.ds((q % 8) * 16, 16)] = (
                        (v0 + v1) * 0.5)
                return rc

            lax.fori_loop(0, _B, row_body, 0, unroll=False)

    for t in range(3):
        issue_idx(t, t)
    for t in range(3):
        wait_idx(t, t)
        issue_gather(t, t)
    issue_idx(3, 3)

    def step(u, carry):
        for k in range(4):
            t = 4 * u + k
            k3 = (k + 3) % 4
            k2 = k % 2
            wait_gather(t, k)
            issue_idx(t + 4, k)
            wait_idx(t + 3, k3)
            issue_gather(t + 3, k3)
            wait_scatter(t - 2, k2)
            compute(t, k, k2)
            issue_scatter(t, k2)
        return carry

    lax.fori_loop(0, _T // 4, step, 0, unroll=False)

    wait_scatter(_T - 2, 0)
    wait_scatter(_T - 1, 1)


_TOPR = 2000


def _tc_top_body(x_ref, o_ref):
    o_ref[...] = x_ref[...]


_tc_top = pl.pallas_call(
    _tc_top_body,
    grid=(_N // _TOPR,),
    in_specs=[pl.BlockSpec((_TOPR, _D), lambda i: (i, 0))],
    out_specs=pl.BlockSpec((_TOPR, _D), lambda i: (i, 0)),
    out_shape=jax.ShapeDtypeStruct((_N + _E, _D), jnp.float32),
)

_MBI = 500


def _tc_mid_body(o_in_ref, m_ref, o_ref):
    del o_in_ref
    m = m_ref[...]
    m = m.reshape(_MBI // 2, 2, 8, 128)
    m = jnp.swapaxes(m, 1, 2)
    o_ref[...] = m.reshape(_MBI * 4, _D)


_tc_mid = pl.pallas_call(
    _tc_mid_body,
    grid=(_MIDI // _MBI,),
    in_specs=[
        pl.BlockSpec((8, 128), lambda i: (0, 0)),
        pl.BlockSpec((_MBI, 8, 128), lambda i: (i, 0, 0)),
    ],
    out_specs=pl.BlockSpec((_MBI * 4, _D), lambda i: (_N // (_MBI * 4) + i, 0)),
    out_shape=jax.ShapeDtypeStruct((_N + _E, _D), jnp.float32),
    input_output_aliases={0: 0},
)


def kernel(inputs, pool_idx):
    idx = pool_idx.reshape(_NCHT, _CHB).astype(jnp.int32)
    out0 = _tc_top(inputs)
    mid3 = _sc_edges(inputs, idx)
    return _tc_mid(out0, mid3)
